# baseline (speedup 1.0000x reference)
import functools

import jax
import jax.numpy as jnp
from jax import lax
from jax.experimental import pallas as pl
from jax.experimental.pallas import tpu as pltpu
from jax.experimental.pallas import tpu_sc as plsc

N_STATIC = 4096
MAX_D = 512
NUM_BUCKETS = 2 * MAX_D - 1
GW = 2 * N_STATIC
NC, NS, L = 2, 16, 16
NW = NC * NS
SLABS = N_STATIC // 8
SLABS_PER_W = SLABS // NW


def _sc_body(t_hbm, out_hbm, t_vmem, g2_vmem, sem):
    wid = lax.axis_index("s") * NC + lax.axis_index("c")
    a = wid % 16
    b = wid // 16
    off = 127 - 8 * a
    pltpu.sync_copy(t_hbm, t_vmem)

    vhi = t_vmem[pl.ds(1024, L)]
    vlo = t_vmem[pl.ds(1040, L)]
    for r in range(8):
        def _left(c, carry, _r=r):
            g2_vmem[_r, pl.ds(c * L, L)] = vhi
            return carry

        lax.fori_loop(0, 3456 // L, _left, 0)

        def _right(c, carry, _r=r):
            g2_vmem[_r, pl.ds(4736 + c * L, L)] = vlo
            return carry

        lax.fori_loop(0, (GW - 4736) // L, _right, 0)

        def _mid(c, carry, _r=r):
            q = 3456 + c * L + lax.broadcasted_iota(jnp.int32, (L,), 0)
            d = jnp.clip((N_STATIC - 1) - (q + off) + _r,
                         -(MAX_D - 1), MAX_D - 1)
            g2_vmem[_r, pl.ds(3456 + c * L, L)] = plsc.load_gather(
                t_vmem, [d + (MAX_D - 1)])
            return carry

        lax.fori_loop(0, (4736 - 3456) // L, _mid, 0)

    def slab(mm, carry):
        m = b * 16 + mm
        k = 16 * m + a
        s_m = pl.multiple_of(3968 - 128 * m, 128)
        pltpu.sync_copy(g2_vmem.at[:, pl.ds(s_m, N_STATIC)],
                        out_hbm.at[pl.ds(pl.multiple_of(8 * k, 8), 8), :])
        return carry

    lax.fori_loop(0, SLABS_PER_W, slab, 0)


@jax.jit
def _rel_pos_bias(t_pad):
    kern = pl.kernel(
        _sc_body,
        out_type=jax.ShapeDtypeStruct((N_STATIC, N_STATIC), jnp.float32),
        mesh=plsc.VectorSubcoreMesh(core_axis_name="c", subcore_axis_name="s"),
        scratch_types=[
            pltpu.VMEM((1056,), jnp.float32),
            pltpu.VMEM((8, GW), jnp.float32),
            pltpu.SemaphoreType.DMA,
        ],
        compiler_params=pltpu.CompilerParams(needs_layout_passes=False),
    )
    return kern(t_pad)


def kernel(N, emb_weight):
    t = emb_weight.reshape(-1)
    t_pad = jnp.concatenate([
        t, jnp.zeros((1,), jnp.float32),
        jnp.broadcast_to(t[NUM_BUCKETS - 1], (16,)),
        jnp.broadcast_to(t[0], (16,)),
    ])
    return _rel_pos_bias(t_pad)


# speedup vs baseline: 1.2614x; 1.2614x over previous
import functools

import jax
import jax.numpy as jnp
from jax import lax
from jax.experimental import pallas as pl
from jax.experimental.pallas import tpu as pltpu
from jax.experimental.pallas import tpu_sc as plsc

N_STATIC = 4096
MAX_D = 512
NUM_BUCKETS = 2 * MAX_D - 1
GW = 2 * N_STATIC
NC, NS, L = 2, 16, 16
NW = NC * NS
SLABS = N_STATIC // 8
SLABS_PER_W = SLABS // NW


def _sc_body(t_hbm, out_hbm, t_vmem, g2_vmem, sem):
    wid = lax.axis_index("s") * NC + lax.axis_index("c")
    a = wid % 16
    b = wid // 16
    off = 127 - 8 * a
    pltpu.sync_copy(t_hbm, t_vmem)

    vhi = t_vmem[pl.ds(1024, L)]
    vlo = t_vmem[pl.ds(1040, L)]
    left_lo = jnp.where(b == 0, 2048 // L, 0)
    right_hi = jnp.where(b == 0, (8064 - 4736) // L, (6016 - 4736) // L)

    for r in range(8):
        @plsc.parallel_loop(left_lo, 3456 // L, unroll=8)
        def _left(c, _r=r):
            g2_vmem[_r, pl.ds(c * L, L)] = vhi

        @plsc.parallel_loop(0, right_hi, unroll=8)
        def _right(c, _r=r):
            g2_vmem[_r, pl.ds(4736 + c * L, L)] = vlo

        @plsc.parallel_loop(0, (4736 - 3456) // L, unroll=4)
        def _mid(c, _r=r):
            q = 3456 + c * L + lax.broadcasted_iota(jnp.int32, (L,), 0)
            d = jnp.clip((N_STATIC - 1) - (q + off) + _r,
                         -(MAX_D - 1), MAX_D - 1)
            g2_vmem[_r, pl.ds(3456 + c * L, L)] = plsc.load_gather(
                t_vmem, [d + (MAX_D - 1)])

    def _slab_copy(m):
        k = 16 * (b * 16 + m) + a
        s_m = pl.multiple_of(3968 - 128 * (b * 16 + m), 128)
        return pltpu.make_async_copy(
            g2_vmem.at[:, pl.ds(s_m, N_STATIC)],
            out_hbm.at[pl.ds(pl.multiple_of(8 * k, 8), 8), :],
            sem)

    _slab_copy(0).start()

    def slab(m, carry):
        _slab_copy(m).start()
        _slab_copy(m - 1).wait()
        return carry

    lax.fori_loop(1, SLABS_PER_W, slab, 0)
    _slab_copy(SLABS_PER_W - 1).wait()


@jax.jit
def _rel_pos_bias(t_pad):
    kern = pl.kernel(
        _sc_body,
        out_type=jax.ShapeDtypeStruct((N_STATIC, N_STATIC), jnp.float32),
        mesh=plsc.VectorSubcoreMesh(core_axis_name="c", subcore_axis_name="s"),
        scratch_types=[
            pltpu.VMEM((1056,), jnp.float32),
            pltpu.VMEM((8, GW), jnp.float32),
            pltpu.SemaphoreType.DMA,
        ],
        compiler_params=pltpu.CompilerParams(needs_layout_passes=False),
    )
    return kern(t_pad)


def kernel(N, emb_weight):
    t = emb_weight.reshape(-1)
    t_pad = jnp.concatenate([
        t, jnp.zeros((1,), jnp.float32),
        jnp.broadcast_to(t[NUM_BUCKETS - 1], (16,)),
        jnp.broadcast_to(t[0], (16,)),
    ])
    return _rel_pos_bias(t_pad)
